# baseline (speedup 1.0000x reference)
import functools

import jax
import jax.numpy as jnp
from jax import lax
from jax.experimental import pallas as pl
from jax.experimental.pallas import tpu as pltpu
from jax.experimental.pallas import tpu_sc as plsc

N_TOKENS = 4096
IN_FEAT = 1024
OUT_FEAT = 1024
NUM_EXPERT = 8

NUM_CORES = 2
NUM_SUBCORES = 16
NW = NUM_CORES * NUM_SUBCORES
ROWS_PER_W = N_TOKENS // NW
CHUNK = 32
CHUNKS = ROWS_PER_W // CHUNK
NBUF = 2

BLOCK_M = 512
M_TILES = N_TOKENS // BLOCK_M
T_STEPS = M_TILES + NUM_EXPERT - 1


def _routing_metadata(gate):
    g = gate.astype(jnp.int32)
    eids = jnp.arange(NUM_EXPERT, dtype=jnp.int32)
    oh = (g[:, None] == eids[None, :]).astype(jnp.int32)
    counts = jnp.sum(oh, axis=0)
    csum = jnp.cumsum(oh, axis=0)
    within = jnp.sum(oh * csum, axis=1) - 1
    ends = jnp.cumsum(counts)
    offs = ends - counts
    rank = jnp.sum(oh * offs[None, :], axis=1) + within

    first = offs // BLOCK_M
    last = (ends - 1) // BLOCK_M
    ntile = jnp.where(counts > 0, last - first + 1, 0)
    tends = jnp.cumsum(ntile)
    tstarts = tends - ntile
    total = tends[NUM_EXPERT - 1]

    t = jnp.arange(T_STEPS, dtype=jnp.int32)
    e_of_t = jnp.minimum(
        jnp.sum((tends[None, :] <= t[:, None]).astype(jnp.int32), axis=1),
        NUM_EXPERT - 1)
    eoh = (e_of_t[:, None] == eids[None, :]).astype(jnp.int32)
    m_of_t = (jnp.sum(eoh * first[None, :], axis=1)
              + t - jnp.sum(eoh * tstarts[None, :], axis=1))
    lo_t = jnp.clip(jnp.sum(eoh * offs[None, :], axis=1) - m_of_t * BLOCK_M,
                    0, BLOCK_M)
    hi_t = jnp.clip(jnp.sum(eoh * ends[None, :], axis=1) - m_of_t * BLOCK_M,
                    0, BLOCK_M)
    valid = t < total
    e_of_t = jnp.where(valid, e_of_t, NUM_EXPERT - 1)
    m_of_t = jnp.where(valid, m_of_t, M_TILES - 1)
    lo_t = jnp.where(valid, lo_t, 0)
    hi_t = jnp.where(valid, hi_t, 0)
    chg = jnp.concatenate(
        [jnp.zeros((1,), jnp.int32),
         (e_of_t[1:] != e_of_t[:-1]).astype(jnp.int32)])
    cnt_t = jnp.cumsum(chg)
    return (rank.astype(jnp.int32), e_of_t.astype(jnp.int32),
            m_of_t.astype(jnp.int32), lo_t.astype(jnp.int32),
            hi_t.astype(jnp.int32), cnt_t.astype(jnp.int32))


@functools.lru_cache(maxsize=None)
def _sc_kernels():
    mesh = plsc.VectorSubcoreMesh(core_axis_name="c", subcore_axis_name="s")
    scratch = [
        pltpu.VMEM((CHUNKS, CHUNK), jnp.int32),
        [pltpu.VMEM((CHUNK, IN_FEAT), jnp.float32) for _ in range(NBUF)],
        [pltpu.SemaphoreType.DMA for _ in range(NBUF)],
        [pltpu.SemaphoreType.DMA for _ in range(NBUF)],
    ]

    @functools.partial(
        pl.kernel, mesh=mesh,
        out_type=jax.ShapeDtypeStruct((N_TOKENS, IN_FEAT), jnp.float32),
        scratch_types=scratch,
    )
    def _sc_dispatch(x_hbm, idx_hbm, out_hbm, idx_v, rows, in_sems, out_sems):
        wid = lax.axis_index("s") * NUM_CORES + lax.axis_index("c")
        base = wid * ROWS_PER_W
        pltpu.sync_copy(idx_hbm.at[wid], idx_v)

        def load(j, b):
            return pltpu.async_copy(
                x_hbm.at[pl.ds(base + j * CHUNK, CHUNK)], rows[b], in_sems[b])

        def store(j, b):
            return pltpu.async_copy(
                rows[b], out_hbm.at[idx_v.at[j]], out_sems[b])

        loads = [None] * NBUF
        stores = [None] * NBUF
        loads[0] = load(0, 0)
        for j in range(CHUNKS):
            b = j % NBUF
            nb = (j + 1) % NBUF
            if j + 1 < CHUNKS:
                if stores[nb] is not None:
                    stores[nb].wait()
                    stores[nb] = None
                loads[nb] = load(j + 1, nb)
            loads[b].wait()
            stores[b] = store(j, b)
        for b in range(NBUF):
            if stores[b] is not None:
                stores[b].wait()

    @functools.partial(
        pl.kernel, mesh=mesh,
        out_type=jax.ShapeDtypeStruct((N_TOKENS, OUT_FEAT), jnp.float32),
        scratch_types=scratch,
    )
    def _sc_combine(y_hbm, idx_hbm, out_hbm, idx_v, rows, in_sems, out_sems):
        wid = lax.axis_index("s") * NUM_CORES + lax.axis_index("c")
        base = wid * ROWS_PER_W
        pltpu.sync_copy(idx_hbm.at[wid], idx_v)

        def load(j, b):
            return pltpu.async_copy(
                y_hbm.at[idx_v.at[j]], rows[b], in_sems[b])

        def store(j, b):
            return pltpu.async_copy(
                rows[b], out_hbm.at[pl.ds(base + j * CHUNK, CHUNK)],
                out_sems[b])

        loads = [None] * NBUF
        stores = [None] * NBUF
        loads[0] = load(0, 0)
        for j in range(CHUNKS):
            b = j % NBUF
            nb = (j + 1) % NBUF
            if j + 1 < CHUNKS:
                if stores[nb] is not None:
                    stores[nb].wait()
                    stores[nb] = None
                loads[nb] = load(j + 1, nb)
            loads[b].wait()
            stores[b] = store(j, b)
        for b in range(NBUF):
            if stores[b] is not None:
                stores[b].wait()

    return _sc_dispatch, _sc_combine


def _mm_body(e_ref, m_ref, lo_ref, hi_ref, cnt_ref, x_ref, w_ref, o_ref):
    t = pl.program_id(0)
    tm1 = jnp.maximum(t - 1, 0)
    rows = lax.broadcasted_iota(jnp.int32, (BLOCK_M, 1), 0)
    mask = ((rows >= lo_ref[t]) & (rows < hi_ref[t])).astype(jnp.bfloat16)
    xm = x_ref[...].astype(jnp.bfloat16) * mask
    partial = lax.dot_general(
        xm, w_ref[e_ref[t]].astype(jnp.bfloat16),
        dimension_numbers=(((1,), (1,)), ((), ())),
        preferred_element_type=jnp.float32,
    )
    first_visit = jnp.logical_or(t == 0, m_ref[t] != m_ref[tm1])

    @pl.when(first_visit)
    def _init():
        o_ref[...] = partial

    @pl.when(jnp.logical_not(first_visit))
    def _acc():
        o_ref[...] += partial


def _grouped_matmul(x_sorted, weight, e_of_t, m_of_t, lo_t, hi_t, cnt_t):
    grid_spec = pltpu.PrefetchScalarGridSpec(
        num_scalar_prefetch=5,
        grid=(T_STEPS,),
        in_specs=[
            pl.BlockSpec((BLOCK_M, IN_FEAT),
                         lambda t, e, m, lo, hi, cnt: (m[t], 0)),
            pl.BlockSpec((NUM_EXPERT, OUT_FEAT, IN_FEAT),
                         lambda t, e, m, lo, hi, cnt: (0, 0, 0)),
        ],
        out_specs=pl.BlockSpec((BLOCK_M, OUT_FEAT),
                               lambda t, e, m, lo, hi, cnt: (m[t], 0)),
    )
    return pl.pallas_call(
        _mm_body,
        grid_spec=grid_spec,
        out_shape=jax.ShapeDtypeStruct((N_TOKENS, OUT_FEAT), jnp.float32),
    )(e_of_t, m_of_t, lo_t, hi_t, cnt_t, x_sorted, weight)


def kernel(inp, gate, weight):
    rank, e_of_t, m_of_t, lo_t, hi_t, cnt_t = _routing_metadata(gate)
    rank3 = rank.reshape(NW, CHUNKS, CHUNK)
    sc_dispatch, sc_combine = _sc_kernels()
    x_sorted = sc_dispatch(inp, rank3)
    y_sorted = _grouped_matmul(x_sorted, weight, e_of_t, m_of_t, lo_t, hi_t,
                               cnt_t)
    return sc_combine(y_sorted, rank3)


# speedup vs baseline: 1.0552x; 1.0552x over previous
import functools

import jax
import jax.numpy as jnp
from jax import lax
from jax.experimental import pallas as pl
from jax.experimental.pallas import tpu as pltpu
from jax.experimental.pallas import tpu_sc as plsc

N_TOKENS = 4096
IN_FEAT = 1024
OUT_FEAT = 1024
NUM_EXPERT = 8

NUM_CORES = 2
NUM_SUBCORES = 16
NW = NUM_CORES * NUM_SUBCORES
ROWS_PER_W = N_TOKENS // NW
CHUNK = 32
CHUNKS = ROWS_PER_W // CHUNK
NBUF = 2

BLOCK_M = 512
M_TILES = N_TOKENS // BLOCK_M
T_STEPS = M_TILES + NUM_EXPERT - 1


def _routing_metadata(gate):
    g = gate.astype(jnp.int32)
    eids = jnp.arange(NUM_EXPERT, dtype=jnp.int32)
    ohf = (g[:, None] == eids[None, :]).astype(jnp.float32)
    oh3 = ohf.reshape(N_TOKENS // 128, 128, NUM_EXPERT)
    tril = jnp.tril(jnp.ones((128, 128), jnp.float32))
    local_incl = jnp.einsum('lk,bke->ble', tril, oh3)
    blocksums = jnp.sum(oh3, axis=1)
    prefix = jnp.cumsum(blocksums, axis=0) - blocksums
    csum = (local_incl + prefix[:, None, :]).reshape(N_TOKENS, NUM_EXPERT)
    within = jnp.sum(ohf * csum, axis=1).astype(jnp.int32) - 1
    counts = jnp.sum(blocksums, axis=0).astype(jnp.int32)
    ends = jnp.cumsum(counts)
    offs = ends - counts
    oh = ohf.astype(jnp.int32)
    rank = jnp.sum(oh * offs[None, :], axis=1) + within

    first = offs // BLOCK_M
    last = (ends - 1) // BLOCK_M
    ntile = jnp.where(counts > 0, last - first + 1, 0)
    tends = jnp.cumsum(ntile)
    tstarts = tends - ntile
    total = tends[NUM_EXPERT - 1]

    t = jnp.arange(T_STEPS, dtype=jnp.int32)
    e_of_t = jnp.minimum(
        jnp.sum((tends[None, :] <= t[:, None]).astype(jnp.int32), axis=1),
        NUM_EXPERT - 1)
    eoh = (e_of_t[:, None] == eids[None, :]).astype(jnp.int32)
    m_of_t = (jnp.sum(eoh * first[None, :], axis=1)
              + t - jnp.sum(eoh * tstarts[None, :], axis=1))
    lo_t = jnp.clip(jnp.sum(eoh * offs[None, :], axis=1) - m_of_t * BLOCK_M,
                    0, BLOCK_M)
    hi_t = jnp.clip(jnp.sum(eoh * ends[None, :], axis=1) - m_of_t * BLOCK_M,
                    0, BLOCK_M)
    valid = t < total
    e_of_t = jnp.where(valid, e_of_t, NUM_EXPERT - 1)
    m_of_t = jnp.where(valid, m_of_t, M_TILES - 1)
    lo_t = jnp.where(valid, lo_t, 0)
    hi_t = jnp.where(valid, hi_t, 0)
    chg = jnp.concatenate(
        [jnp.zeros((1,), jnp.int32),
         (e_of_t[1:] != e_of_t[:-1]).astype(jnp.int32)])
    cnt_t = jnp.cumsum(chg)
    return (rank.astype(jnp.int32), e_of_t.astype(jnp.int32),
            m_of_t.astype(jnp.int32), lo_t.astype(jnp.int32),
            hi_t.astype(jnp.int32), cnt_t.astype(jnp.int32))


@functools.lru_cache(maxsize=None)
def _sc_kernels():
    mesh = plsc.VectorSubcoreMesh(core_axis_name="c", subcore_axis_name="s")
    scratch = [
        pltpu.VMEM((CHUNKS, CHUNK), jnp.int32),
        [pltpu.VMEM((CHUNK, IN_FEAT), jnp.float32) for _ in range(NBUF)],
        [pltpu.SemaphoreType.DMA for _ in range(NBUF)],
        [pltpu.SemaphoreType.DMA for _ in range(NBUF)],
    ]

    @functools.partial(
        pl.kernel, mesh=mesh,
        out_type=jax.ShapeDtypeStruct((N_TOKENS, IN_FEAT), jnp.float32),
        scratch_types=scratch,
    )
    def _sc_dispatch(x_hbm, idx_hbm, out_hbm, idx_v, rows, in_sems, out_sems):
        wid = lax.axis_index("s") * NUM_CORES + lax.axis_index("c")
        base = wid * ROWS_PER_W
        pltpu.sync_copy(idx_hbm.at[wid], idx_v)

        def load(j, b):
            return pltpu.async_copy(
                x_hbm.at[pl.ds(base + j * CHUNK, CHUNK)], rows[b], in_sems[b])

        def store(j, b):
            return pltpu.async_copy(
                rows[b], out_hbm.at[idx_v.at[j]], out_sems[b])

        loads = [None] * NBUF
        stores = [None] * NBUF
        loads[0] = load(0, 0)
        for j in range(CHUNKS):
            b = j % NBUF
            nb = (j + 1) % NBUF
            if j + 1 < CHUNKS:
                if stores[nb] is not None:
                    stores[nb].wait()
                    stores[nb] = None
                loads[nb] = load(j + 1, nb)
            loads[b].wait()
            stores[b] = store(j, b)
        for b in range(NBUF):
            if stores[b] is not None:
                stores[b].wait()

    @functools.partial(
        pl.kernel, mesh=mesh,
        out_type=jax.ShapeDtypeStruct((N_TOKENS, OUT_FEAT), jnp.float32),
        scratch_types=scratch,
    )
    def _sc_combine(y_hbm, idx_hbm, out_hbm, idx_v, rows, in_sems, out_sems):
        wid = lax.axis_index("s") * NUM_CORES + lax.axis_index("c")
        base = wid * ROWS_PER_W
        pltpu.sync_copy(idx_hbm.at[wid], idx_v)

        def load(j, b):
            return pltpu.async_copy(
                y_hbm.at[idx_v.at[j]], rows[b], in_sems[b])

        def store(j, b):
            return pltpu.async_copy(
                rows[b], out_hbm.at[pl.ds(base + j * CHUNK, CHUNK)],
                out_sems[b])

        loads = [None] * NBUF
        stores = [None] * NBUF
        loads[0] = load(0, 0)
        for j in range(CHUNKS):
            b = j % NBUF
            nb = (j + 1) % NBUF
            if j + 1 < CHUNKS:
                if stores[nb] is not None:
                    stores[nb].wait()
                    stores[nb] = None
                loads[nb] = load(j + 1, nb)
            loads[b].wait()
            stores[b] = store(j, b)
        for b in range(NBUF):
            if stores[b] is not None:
                stores[b].wait()

    return _sc_dispatch, _sc_combine


def _mm_body(e_ref, m_ref, lo_ref, hi_ref, cnt_ref, x_ref, w_ref, o_ref):
    t = pl.program_id(0)
    tm1 = jnp.maximum(t - 1, 0)
    rows = lax.broadcasted_iota(jnp.int32, (BLOCK_M, 1), 0)
    mask = ((rows >= lo_ref[t]) & (rows < hi_ref[t])).astype(jnp.bfloat16)
    xm = x_ref[...].astype(jnp.bfloat16) * mask
    partial = lax.dot_general(
        xm, w_ref[0].astype(jnp.bfloat16),
        dimension_numbers=(((1,), (1,)), ((), ())),
        preferred_element_type=jnp.float32,
    )
    first_visit = jnp.logical_or(t == 0, m_ref[t] != m_ref[tm1])

    @pl.when(first_visit)
    def _init():
        o_ref[...] = partial

    @pl.when(jnp.logical_not(first_visit))
    def _acc():
        o_ref[...] += partial


def _grouped_matmul(x_sorted, weight, e_of_t, m_of_t, lo_t, hi_t, cnt_t):
    grid_spec = pltpu.PrefetchScalarGridSpec(
        num_scalar_prefetch=5,
        grid=(T_STEPS,),
        in_specs=[
            pl.BlockSpec((BLOCK_M, IN_FEAT),
                         lambda t, e, m, lo, hi, cnt: (m[t], 0)),
            pl.BlockSpec((1, OUT_FEAT, IN_FEAT),
                         lambda t, e, m, lo, hi, cnt: (e[t], 0, 0)),
        ],
        out_specs=pl.BlockSpec((BLOCK_M, OUT_FEAT),
                               lambda t, e, m, lo, hi, cnt: (m[t], 0)),
    )
    return pl.pallas_call(
        _mm_body,
        grid_spec=grid_spec,
        out_shape=jax.ShapeDtypeStruct((N_TOKENS, OUT_FEAT), jnp.float32),
    )(e_of_t, m_of_t, lo_t, hi_t, cnt_t, x_sorted, weight)


def kernel(inp, gate, weight):
    rank, e_of_t, m_of_t, lo_t, hi_t, cnt_t = _routing_metadata(gate)
    rank3 = rank.reshape(NW, CHUNKS, CHUNK)
    sc_dispatch, sc_combine = _sc_kernels()
    x_sorted = sc_dispatch(inp, rank3)
    y_sorted = _grouped_matmul(x_sorted, weight, e_of_t, m_of_t, lo_t, hi_t,
                               cnt_t)
    return sc_combine(y_sorted, rank3)


# speedup vs baseline: 1.0571x; 1.0017x over previous
import functools

import jax
import jax.numpy as jnp
from jax import lax
from jax.experimental import pallas as pl
from jax.experimental.pallas import tpu as pltpu
from jax.experimental.pallas import tpu_sc as plsc

N_TOKENS = 4096
IN_FEAT = 1024
OUT_FEAT = 1024
NUM_EXPERT = 8

NUM_CORES = 2
NUM_SUBCORES = 16
NW = NUM_CORES * NUM_SUBCORES
ROWS_PER_W = N_TOKENS // NW
CHUNK = 32
CHUNKS = ROWS_PER_W // CHUNK
NBUF = 2

BLOCK_M = 512
M_TILES = N_TOKENS // BLOCK_M
T_STEPS = M_TILES + NUM_EXPERT - 1


def _routing_metadata(gate):
    g = gate.astype(jnp.int32)
    eids = jnp.arange(NUM_EXPERT, dtype=jnp.int32)
    ohf = (g[:, None] == eids[None, :]).astype(jnp.float32)
    oh3 = ohf.reshape(N_TOKENS // 128, 128, NUM_EXPERT)
    tril = jnp.tril(jnp.ones((128, 128), jnp.float32))
    local_incl = jnp.einsum('lk,bke->ble', tril, oh3)
    blocksums = jnp.sum(oh3, axis=1)
    prefix = jnp.cumsum(blocksums, axis=0) - blocksums
    csum = (local_incl + prefix[:, None, :]).reshape(N_TOKENS, NUM_EXPERT)
    within = jnp.sum(ohf * csum, axis=1).astype(jnp.int32) - 1
    counts = jnp.sum(blocksums, axis=0).astype(jnp.int32)
    ends = jnp.cumsum(counts)
    offs = ends - counts
    oh = ohf.astype(jnp.int32)
    rank = jnp.sum(oh * offs[None, :], axis=1) + within

    first = offs // BLOCK_M
    last = (ends - 1) // BLOCK_M
    ntile = jnp.where(counts > 0, last - first + 1, 0)
    tends = jnp.cumsum(ntile)
    tstarts = tends - ntile
    total = tends[NUM_EXPERT - 1]

    t = jnp.arange(T_STEPS, dtype=jnp.int32)
    e_of_t = jnp.minimum(
        jnp.sum((tends[None, :] <= t[:, None]).astype(jnp.int32), axis=1),
        NUM_EXPERT - 1)
    eoh = (e_of_t[:, None] == eids[None, :]).astype(jnp.int32)
    m_of_t = (jnp.sum(eoh * first[None, :], axis=1)
              + t - jnp.sum(eoh * tstarts[None, :], axis=1))
    lo_t = jnp.clip(jnp.sum(eoh * offs[None, :], axis=1) - m_of_t * BLOCK_M,
                    0, BLOCK_M)
    hi_t = jnp.clip(jnp.sum(eoh * ends[None, :], axis=1) - m_of_t * BLOCK_M,
                    0, BLOCK_M)
    valid = t < total
    e_of_t = jnp.where(valid, e_of_t, NUM_EXPERT - 1)
    m_of_t = jnp.where(valid, m_of_t, M_TILES - 1)
    lo_t = jnp.where(valid, lo_t, 0)
    hi_t = jnp.where(valid, hi_t, 0)
    return (rank.astype(jnp.int32), e_of_t.astype(jnp.int32),
            m_of_t.astype(jnp.int32), lo_t.astype(jnp.int32),
            hi_t.astype(jnp.int32))


@functools.lru_cache(maxsize=None)
def _sc_kernels():
    mesh = plsc.VectorSubcoreMesh(core_axis_name="c", subcore_axis_name="s")
    scratch = [
        pltpu.VMEM((CHUNKS, CHUNK), jnp.int32),
        [pltpu.VMEM((CHUNK, IN_FEAT), jnp.float32) for _ in range(NBUF)],
        [pltpu.SemaphoreType.DMA for _ in range(NBUF)],
        [pltpu.SemaphoreType.DMA for _ in range(NBUF)],
    ]

    @functools.partial(
        pl.kernel, mesh=mesh,
        out_type=jax.ShapeDtypeStruct((N_TOKENS, IN_FEAT), jnp.float32),
        scratch_types=scratch,
    )
    def _sc_dispatch(x_hbm, idx_hbm, out_hbm, idx_v, rows, in_sems, out_sems):
        wid = lax.axis_index("s") * NUM_CORES + lax.axis_index("c")
        base = wid * ROWS_PER_W
        pltpu.sync_copy(idx_hbm.at[wid], idx_v)

        def load(j, b):
            return pltpu.async_copy(
                x_hbm.at[pl.ds(base + j * CHUNK, CHUNK)], rows[b], in_sems[b])

        def store(j, b):
            return pltpu.async_copy(
                rows[b], out_hbm.at[idx_v.at[j]], out_sems[b])

        loads = [None] * NBUF
        stores = [None] * NBUF
        loads[0] = load(0, 0)
        for j in range(CHUNKS):
            b = j % NBUF
            nb = (j + 1) % NBUF
            if j + 1 < CHUNKS:
                if stores[nb] is not None:
                    stores[nb].wait()
                    stores[nb] = None
                loads[nb] = load(j + 1, nb)
            loads[b].wait()
            stores[b] = store(j, b)
        for b in range(NBUF):
            if stores[b] is not None:
                stores[b].wait()

    @functools.partial(
        pl.kernel, mesh=mesh,
        out_type=jax.ShapeDtypeStruct((N_TOKENS, OUT_FEAT), jnp.float32),
        scratch_types=scratch,
    )
    def _sc_combine(y_hbm, idx_hbm, out_hbm, idx_v, rows, in_sems, out_sems):
        wid = lax.axis_index("s") * NUM_CORES + lax.axis_index("c")
        base = wid * ROWS_PER_W
        pltpu.sync_copy(idx_hbm.at[wid], idx_v)

        def load(j, b):
            return pltpu.async_copy(
                y_hbm.at[idx_v.at[j]], rows[b], in_sems[b])

        def store(j, b):
            return pltpu.async_copy(
                rows[b], out_hbm.at[pl.ds(base + j * CHUNK, CHUNK)],
                out_sems[b])

        loads = [None] * NBUF
        stores = [None] * NBUF
        loads[0] = load(0, 0)
        for j in range(CHUNKS):
            b = j % NBUF
            nb = (j + 1) % NBUF
            if j + 1 < CHUNKS:
                if stores[nb] is not None:
                    stores[nb].wait()
                    stores[nb] = None
                loads[nb] = load(j + 1, nb)
            loads[b].wait()
            stores[b] = store(j, b)
        for b in range(NBUF):
            if stores[b] is not None:
                stores[b].wait()

    return _sc_dispatch, _sc_combine


def _mm_body(e_ref, m_ref, lo_ref, hi_ref, x_ref, w_ref, o_ref):
    t = pl.program_id(0)
    tm1 = jnp.maximum(t - 1, 0)
    rows = lax.broadcasted_iota(jnp.int32, (BLOCK_M, 1), 0)
    mask = ((rows >= lo_ref[t]) & (rows < hi_ref[t])).astype(jnp.bfloat16)
    xm = x_ref[...].astype(jnp.bfloat16) * mask
    partial = lax.dot_general(
        xm, w_ref[0].astype(jnp.bfloat16),
        dimension_numbers=(((1,), (1,)), ((), ())),
        preferred_element_type=jnp.float32,
    )
    first_visit = jnp.logical_or(t == 0, m_ref[t] != m_ref[tm1])

    @pl.when(first_visit)
    def _init():
        o_ref[...] = partial

    @pl.when(jnp.logical_not(first_visit))
    def _acc():
        o_ref[...] += partial


def _grouped_matmul(x_sorted, weight, e_of_t, m_of_t, lo_t, hi_t):
    grid_spec = pltpu.PrefetchScalarGridSpec(
        num_scalar_prefetch=4,
        grid=(T_STEPS,),
        in_specs=[
            pl.BlockSpec((BLOCK_M, IN_FEAT),
                         lambda t, e, m, lo, hi: (m[t], 0)),
            pl.BlockSpec((1, OUT_FEAT, IN_FEAT),
                         lambda t, e, m, lo, hi: (e[t], 0, 0)),
        ],
        out_specs=pl.BlockSpec((BLOCK_M, OUT_FEAT),
                               lambda t, e, m, lo, hi: (m[t], 0)),
    )
    return pl.pallas_call(
        _mm_body,
        grid_spec=grid_spec,
        out_shape=jax.ShapeDtypeStruct((N_TOKENS, OUT_FEAT), jnp.float32),
    )(e_of_t, m_of_t, lo_t, hi_t, x_sorted, weight)


def kernel(inp, gate, weight):
    rank, e_of_t, m_of_t, lo_t, hi_t = _routing_metadata(gate)
    rank3 = rank.reshape(NW, CHUNKS, CHUNK)
    sc_dispatch, sc_combine = _sc_kernels()
    x_sorted = sc_dispatch(inp, rank3)
    y_sorted = _grouped_matmul(x_sorted, weight, e_of_t, m_of_t, lo_t, hi_t)
    return sc_combine(y_sorted, rank3)
